# baseline (speedup 1.0000x reference)
import functools
import math

import jax
import jax.numpy as jnp
from jax.experimental import pallas as pl
from jax.experimental.pallas import tpu as pltpu

_BB = 2


def _attn_kernel(x_ref, w_ref, o_ref, *, dim_attn):
    bb, S, D = x_ref.shape
    a = dim_attn

    x2d = x_ref[...].reshape(bb * S, D)
    qkv = jnp.dot(x2d, w_ref[...], preferred_element_type=jnp.float32)

    q = qkv[:, :a].astype(jnp.bfloat16).reshape(bb, S, a)
    k = qkv[:, a:2 * a].astype(jnp.bfloat16).reshape(bb, S, a)
    v = qkv[:, 2 * a:].astype(jnp.bfloat16).reshape(bb, S, D)

    s = jnp.einsum("bqa,bka->bqk", q, k, preferred_element_type=jnp.float32)
    m = jnp.max(s, axis=-1, keepdims=True)
    e = jnp.exp(s - m)
    denom = jnp.sum(e, axis=-1, keepdims=True)
    o = jnp.einsum("bqk,bkd->bqd", e.astype(jnp.bfloat16), v,
                   preferred_element_type=jnp.float32)
    o_ref[...] = o * pl.reciprocal(denom, approx=True)


def kernel(x, wq, wk, wv):
    B, S, D = x.shape
    A = wq.shape[1]
    scale = jnp.float32(1.0 / math.sqrt(A))

    wqkv = jnp.concatenate([wq * scale, wk, wv], axis=1).astype(jnp.bfloat16)
    x_bf = x.astype(jnp.bfloat16)

    bb = _BB
    while B % bb:
        bb //= 2

    flops = 2 * B * (S * D * (2 * A + D) + S * S * A + S * S * D)
    bytes_accessed = 2 * (x.size + wqkv.size) + 4 * B * S * D

    body = functools.partial(_attn_kernel, dim_attn=A)
    return pl.pallas_call(
        body,
        out_shape=jax.ShapeDtypeStruct((B, S, D), jnp.float32),
        grid=(B // bb,),
        in_specs=[
            pl.BlockSpec((bb, S, D), lambda b: (b, 0, 0)),
            pl.BlockSpec((D, 2 * A + D), lambda b: (0, 0)),
        ],
        out_specs=pl.BlockSpec((bb, S, D), lambda b: (b, 0, 0)),
        compiler_params=pltpu.CompilerParams(
            dimension_semantics=("parallel",)),
        cost_estimate=pl.CostEstimate(
            flops=flops, transcendentals=B * S * S,
            bytes_accessed=bytes_accessed),
    )(x_bf, wqkv)


# speedup vs baseline: 1.2281x; 1.2281x over previous
import functools
import math

import jax
import jax.numpy as jnp
from jax.experimental import pallas as pl
from jax.experimental.pallas import tpu as pltpu

_BB = 2


def _attn_kernel(x_ref, w_ref, o_ref, *, dim_attn):
    bb, S, D = x_ref.shape
    a = dim_attn

    x2d = x_ref[...].reshape(bb * S, D)
    qkv = jnp.dot(x2d, w_ref[...], preferred_element_type=jnp.float32)

    q = qkv[:, :a].reshape(bb, S, a)
    k = qkv[:, a:2 * a].reshape(bb, S, a)
    v = qkv[:, 2 * a:].reshape(bb, S, D)

    s = jnp.einsum("bqa,bka->bqk", q, k, preferred_element_type=jnp.float32)
    m = jnp.max(s, axis=-1, keepdims=True)
    e = jnp.exp(s - m)
    denom = jnp.sum(e, axis=-1, keepdims=True)
    o = jnp.einsum("bqk,bkd->bqd", e, v,
                   preferred_element_type=jnp.float32)
    o_ref[...] = o * pl.reciprocal(denom, approx=True)


def kernel(x, wq, wk, wv):
    B, S, D = x.shape
    A = wq.shape[1]
    scale = jnp.float32(1.0 / math.sqrt(A))

    wqkv = jnp.concatenate([wq * scale, wk, wv], axis=1)
    x_bf = x

    bb = _BB
    while B % bb:
        bb //= 2

    flops = 2 * B * (S * D * (2 * A + D) + S * S * A + S * S * D)
    bytes_accessed = 4 * (x.size + wqkv.size + B * S * D)

    body = functools.partial(_attn_kernel, dim_attn=A)
    return pl.pallas_call(
        body,
        out_shape=jax.ShapeDtypeStruct((B, S, D), jnp.float32),
        grid=(B // bb,),
        in_specs=[
            pl.BlockSpec((bb, S, D), lambda b: (b, 0, 0)),
            pl.BlockSpec((D, 2 * A + D), lambda b: (0, 0)),
        ],
        out_specs=pl.BlockSpec((bb, S, D), lambda b: (b, 0, 0)),
        compiler_params=pltpu.CompilerParams(
            dimension_semantics=("parallel",)),
        cost_estimate=pl.CostEstimate(
            flops=flops, transcendentals=B * S * S,
            bytes_accessed=bytes_accessed),
    )(x_bf, wqkv)
